# Initial kernel scaffold; baseline (speedup 1.0000x reference)
import jax
import jax.numpy as jnp
from jax.experimental import pallas as pl


def kernel(inputs, table):
    raise NotImplementedError("write your pallas kernel here")


# speedup vs baseline: 9.6378x; 9.6378x over previous
import jax
import jax.numpy as jnp
from jax import lax
from jax.experimental import pallas as pl
from jax.experimental.pallas import tpu as pltpu
from jax.experimental.pallas import tpu_sc as plsc

_NC, _NS, _L = 2, 16, 16
_NW = _NC * _NS

_BATCH, _HIST = 16384, 200
_N64 = _BATCH * _HIST
_N32 = 2 * _N64
_WPW = _N32 // _NW
_CHUNK = 20480
_NCHUNK = _WPW // _CHUNK
_TAB32 = 240


def _lookup_body(in_hbm, tab_hbm, out_hbm, tab_v, in_v, out_v):
    i32 = jnp.int32
    wid = lax.axis_index("s") * i32(_NC) + lax.axis_index("c")
    base = wid * i32(_WPW)
    pltpu.sync_copy(tab_hbm, tab_v)
    parity = lax.iota(i32, _L) & i32(1)

    for c in range(_NCHUNK):
        off = base + i32(c * _CHUNK)
        pltpu.sync_copy(in_hbm.at[pl.ds(off, _CHUNK)], in_v)

        @plsc.parallel_loop(i32(0), i32(_CHUNK // _L), i32(1), unroll=8)
        def word_body(i):
            j = i * i32(_L)
            w = in_v[pl.ds(j, _L)]
            out_v[pl.ds(j, _L)] = plsc.load_gather(
                tab_v, [w * i32(2) + parity])

        pltpu.sync_copy(out_v, out_hbm.at[pl.ds(off, _CHUNK)])


def kernel(inputs, table):
    in32 = lax.bitcast_convert_type(inputs, jnp.int32).reshape(_N32)
    tab32 = lax.bitcast_convert_type(table, jnp.int32).reshape(_TAB32)
    mesh = plsc.VectorSubcoreMesh(
        core_axis_name="c", subcore_axis_name="s",
        num_cores=_NC, num_subcores=_NS)
    out32 = pl.kernel(
        _lookup_body,
        out_type=jax.ShapeDtypeStruct((_N32,), jnp.int32),
        mesh=mesh,
        scratch_types=[
            pltpu.VMEM((_TAB32,), jnp.int32),
            pltpu.VMEM((_CHUNK,), jnp.int32),
            pltpu.VMEM((_CHUNK,), jnp.int32),
        ],
        compiler_params=pltpu.CompilerParams(needs_layout_passes=False),
    )(in32, tab32)
    return lax.bitcast_convert_type(
        out32.reshape(_BATCH, _HIST, 2), jnp.int64)


# speedup vs baseline: 122.8918x; 12.7510x over previous
import jax
import jax.numpy as jnp
from jax import lax
from jax.experimental import pallas as pl
from jax.experimental.pallas import tpu as pltpu
from jax.experimental.pallas import tpu_sc as plsc

_NC, _NS, _L = 2, 16, 16
_NW = _NC * _NS

_BATCH, _HIST = 16384, 200
_N = _BATCH * _HIST
_WPW = _N // _NW
_CHUNK = 20480
_NCHUNK = _WPW // _CHUNK
_TAB = 120


def _lookup_body(in_hbm, tab_hbm, out_hbm, tab_v, in_v, out_v):
    i32 = jnp.int32
    wid = lax.axis_index("s") * i32(_NC) + lax.axis_index("c")
    base = wid * i32(_WPW)
    pltpu.sync_copy(tab_hbm, tab_v)

    for c in range(_NCHUNK):
        off = base + i32(c * _CHUNK)
        pltpu.sync_copy(in_hbm.at[pl.ds(off, _CHUNK)], in_v)

        @plsc.parallel_loop(i32(0), i32(_CHUNK // _L), i32(1), unroll=8)
        def word_body(i):
            j = i * i32(_L)
            out_v[pl.ds(j, _L)] = plsc.load_gather(tab_v, [in_v[pl.ds(j, _L)]])

        pltpu.sync_copy(out_v, out_hbm.at[pl.ds(off, _CHUNK)])


def kernel(inputs, table):
    in32 = inputs.astype(jnp.int32).reshape(_N)
    tab32 = table.astype(jnp.int32)
    mesh = plsc.VectorSubcoreMesh(
        core_axis_name="c", subcore_axis_name="s",
        num_cores=_NC, num_subcores=_NS)
    out32 = pl.kernel(
        _lookup_body,
        out_type=jax.ShapeDtypeStruct((_N,), jnp.int32),
        mesh=mesh,
        scratch_types=[
            pltpu.VMEM((_TAB,), jnp.int32),
            pltpu.VMEM((_CHUNK,), jnp.int32),
            pltpu.VMEM((_CHUNK,), jnp.int32),
        ],
        compiler_params=pltpu.CompilerParams(needs_layout_passes=False),
    )(in32, tab32)
    return out32.reshape(_BATCH, _HIST).astype(jnp.int64)


# speedup vs baseline: 166.8944x; 1.3581x over previous
import jax
import jax.numpy as jnp
from jax import lax
from jax.experimental import pallas as pl
from jax.experimental.pallas import tpu as pltpu
from jax.experimental.pallas import tpu_sc as plsc

_NC, _NS, _L = 2, 16, 16
_NW = _NC * _NS

_BATCH, _HIST = 16384, 200
_N = _BATCH * _HIST
_WPW = _N // _NW
_CHUNK = 20480
_NCHUNK = _WPW // _CHUNK
_TAB = 120


def _lookup_body(in_hbm, tab_hbm, out_hbm, tab_v, in_v, out_v):
    i32 = jnp.int32
    wid = lax.axis_index("s") * i32(_NC) + lax.axis_index("c")
    base = wid * i32(_WPW)
    pltpu.sync_copy(tab_hbm, tab_v)

    for c in range(_NCHUNK):
        off = base + i32(c * _CHUNK)
        pltpu.sync_copy(in_hbm.at[pl.ds(off, _CHUNK)], in_v)

        @plsc.parallel_loop(i32(0), i32(_CHUNK // _L), i32(1), unroll=8)
        def word_body(i):
            j = i * i32(_L)
            out_v[pl.ds(j, _L)] = plsc.load_gather(tab_v, [in_v[pl.ds(j, _L)]])

        pltpu.sync_copy(out_v, out_hbm.at[pl.ds(off, _CHUNK)])


def kernel(inputs, table):
    in32 = inputs.astype(jnp.int32).T.reshape(_N)
    tab32 = table.astype(jnp.int32)
    mesh = plsc.VectorSubcoreMesh(
        core_axis_name="c", subcore_axis_name="s",
        num_cores=_NC, num_subcores=_NS)
    out32 = pl.kernel(
        _lookup_body,
        out_type=jax.ShapeDtypeStruct((_N,), jnp.int32),
        mesh=mesh,
        scratch_types=[
            pltpu.VMEM((_TAB,), jnp.int32),
            pltpu.VMEM((_CHUNK,), jnp.int32),
            pltpu.VMEM((_CHUNK,), jnp.int32),
        ],
        compiler_params=pltpu.CompilerParams(needs_layout_passes=False),
    )(in32, tab32)
    outu = lax.bitcast_convert_type(out32, jnp.uint32).astype(jnp.uint64)
    return lax.bitcast_convert_type(outu, jnp.int64).reshape(_HIST, _BATCH).T


# speedup vs baseline: 180.0846x; 1.0790x over previous
import jax
import jax.numpy as jnp
from jax import lax
from jax.experimental import pallas as pl
from jax.experimental.pallas import tpu as pltpu
from jax.experimental.pallas import tpu_sc as plsc

_NC, _NS, _L = 2, 16, 16
_NW = _NC * _NS

_BATCH, _HIST = 16384, 200
_N = _BATCH * _HIST
_WPW = _N // _NW
_CHUNK = 20480
_NCHUNK = _WPW // _CHUNK
_TAB = 120


def _lookup_body(in_hbm, tab_hbm, out_hbm, tab_v, in_v, out_v):
    i32 = jnp.int32
    wid = lax.axis_index("s") * i32(_NC) + lax.axis_index("c")
    base = wid * i32(_WPW)
    pltpu.sync_copy(tab_hbm, tab_v)

    for c in range(_NCHUNK):
        off = base + i32(c * _CHUNK)
        pltpu.sync_copy(in_hbm.at[pl.ds(off, _CHUNK)], in_v)

        @plsc.parallel_loop(i32(0), i32(_CHUNK // _L), i32(1), unroll=8)
        def word_body(i):
            j = i * i32(_L)
            out_v[pl.ds(j, _L)] = plsc.load_gather(tab_v, [in_v[pl.ds(j, _L)]])

        pltpu.sync_copy(out_v, out_hbm.at[pl.ds(off, _CHUNK)])


def kernel(inputs, table):
    in32 = (inputs.astype(jnp.int32)
            .reshape(_BATCH // 128, 128, _HIST // 8, 8)
            .transpose(2, 0, 3, 1).reshape(_N))
    tab32 = table.astype(jnp.int32)
    mesh = plsc.VectorSubcoreMesh(
        core_axis_name="c", subcore_axis_name="s",
        num_cores=_NC, num_subcores=_NS)
    out32 = pl.kernel(
        _lookup_body,
        out_type=jax.ShapeDtypeStruct((_N,), jnp.int32),
        mesh=mesh,
        scratch_types=[
            pltpu.VMEM((_TAB,), jnp.int32),
            pltpu.VMEM((_CHUNK,), jnp.int32),
            pltpu.VMEM((_CHUNK,), jnp.int32),
        ],
        compiler_params=pltpu.CompilerParams(needs_layout_passes=False),
    )(in32, tab32)
    out2d = (out32.reshape(_HIST // 8, _BATCH // 128, 8, 128)
             .transpose(1, 3, 0, 2).reshape(_BATCH, _HIST))
    outu = lax.bitcast_convert_type(out2d, jnp.uint32).astype(jnp.uint64)
    return lax.bitcast_convert_type(outu, jnp.int64)


# speedup vs baseline: 180.1568x; 1.0004x over previous
import jax
import jax.numpy as jnp
from jax import lax
from jax.experimental import pallas as pl
from jax.experimental.pallas import tpu as pltpu
from jax.experimental.pallas import tpu_sc as plsc

_NC, _NS, _L = 2, 16, 16
_NW = _NC * _NS

_BATCH, _HIST = 16384, 200
_N = _BATCH * _HIST
_WPW = _N // _NW
_CHUNK = 20480
_NCHUNK = _WPW // _CHUNK
_TAB = 120


def _lookup_body(in_hbm, tab_hbm, out_hbm, tab_v, in_v, out_v):
    i32 = jnp.int32
    wid = lax.axis_index("s") * i32(_NC) + lax.axis_index("c")
    base = wid * i32(_WPW)
    pltpu.sync_copy(tab_hbm, tab_v)

    for c in range(_NCHUNK):
        off = base + i32(c * _CHUNK)
        pltpu.sync_copy(in_hbm.at[pl.ds(off, _CHUNK)], in_v)

        @plsc.parallel_loop(i32(0), i32(_CHUNK // _L), i32(1), unroll=8)
        def word_body(i):
            j = i * i32(_L)
            out_v[pl.ds(j, _L)] = plsc.load_gather(tab_v, [in_v[pl.ds(j, _L)]])

        pltpu.sync_copy(out_v, out_hbm.at[pl.ds(off, _CHUNK)])


def kernel(inputs, table):
    in32 = (lax.bitcast_convert_type(inputs.astype(jnp.uint32), jnp.int32)
            .reshape(_BATCH // 128, 128, _HIST // 8, 8)
            .transpose(2, 0, 3, 1).reshape(_N))
    tab32 = lax.bitcast_convert_type(table.astype(jnp.uint32), jnp.int32)
    mesh = plsc.VectorSubcoreMesh(
        core_axis_name="c", subcore_axis_name="s",
        num_cores=_NC, num_subcores=_NS)
    out32 = pl.kernel(
        _lookup_body,
        out_type=jax.ShapeDtypeStruct((_N,), jnp.int32),
        mesh=mesh,
        scratch_types=[
            pltpu.VMEM((_TAB,), jnp.int32),
            pltpu.VMEM((_CHUNK,), jnp.int32),
            pltpu.VMEM((_CHUNK,), jnp.int32),
        ],
        compiler_params=pltpu.CompilerParams(needs_layout_passes=False),
    )(in32, tab32)
    out2d = (out32.reshape(_HIST // 8, _BATCH // 128, 8, 128)
             .transpose(1, 3, 0, 2).reshape(_BATCH, _HIST))
    outu = lax.bitcast_convert_type(out2d, jnp.uint32).astype(jnp.uint64)
    return lax.bitcast_convert_type(outu, jnp.int64)


# speedup vs baseline: 183.7810x; 1.0201x over previous
import jax
import jax.numpy as jnp
from jax import lax
from jax.experimental import pallas as pl
from jax.experimental.pallas import tpu as pltpu
from jax.experimental.pallas import tpu_sc as plsc

_NC, _NS, _L = 2, 16, 16
_NW = _NC * _NS

_BATCH, _HIST = 16384, 200
_N = _BATCH * _HIST
_WPW = _N // _NW
_CHUNK = 20480
_NCHUNK = _WPW // _CHUNK
_TAB = 120


def _lookup_body(in_hbm, tab_hbm, out_hbm, tab_v,
                 in_v0, in_v1, out_v0, out_v1,
                 sem_i0, sem_i1, sem_o0, sem_o1):
    i32 = jnp.int32
    wid = lax.axis_index("s") * i32(_NC) + lax.axis_index("c")
    base = wid * i32(_WPW)
    pltpu.sync_copy(tab_hbm, tab_v)

    in_bufs, out_bufs = (in_v0, in_v1), (out_v0, out_v1)
    in_sems, out_sems = (sem_i0, sem_i1), (sem_o0, sem_o1)

    def in_dma(c):
        off = base + i32(c * _CHUNK)
        return pltpu.async_copy(
            in_hbm.at[pl.ds(off, _CHUNK)], in_bufs[c % 2], in_sems[c % 2])

    def out_dma(c):
        off = base + i32(c * _CHUNK)
        return pltpu.async_copy(
            out_bufs[c % 2], out_hbm.at[pl.ds(off, _CHUNK)], out_sems[c % 2])

    pending_in = {0: in_dma(0)}
    pending_out = {}
    for c in range(_NCHUNK):
        cur = c % 2
        if c + 1 < _NCHUNK:
            pending_in[c + 1] = in_dma(c + 1)
        pending_in.pop(c).wait()
        if c - 2 in pending_out:
            pending_out.pop(c - 2).wait()
        in_v, out_v = in_bufs[cur], out_bufs[cur]

        @plsc.parallel_loop(i32(0), i32(_CHUNK // _L), i32(1), unroll=8)
        def word_body(i):
            j = i * i32(_L)
            out_v[pl.ds(j, _L)] = plsc.load_gather(tab_v, [in_v[pl.ds(j, _L)]])

        pending_out[c] = out_dma(c)
    for c in sorted(pending_out):
        pending_out.pop(c).wait()


def kernel(inputs, table):
    in32 = (lax.bitcast_convert_type(inputs.astype(jnp.uint32), jnp.int32)
            .reshape(_BATCH // 128, 128, _HIST // 8, 8)
            .transpose(2, 0, 3, 1).reshape(_N))
    tab32 = lax.bitcast_convert_type(table.astype(jnp.uint32), jnp.int32)
    mesh = plsc.VectorSubcoreMesh(
        core_axis_name="c", subcore_axis_name="s",
        num_cores=_NC, num_subcores=_NS)
    out32 = pl.kernel(
        _lookup_body,
        out_type=jax.ShapeDtypeStruct((_N,), jnp.int32),
        mesh=mesh,
        scratch_types=[
            pltpu.VMEM((_TAB,), jnp.int32),
            pltpu.VMEM((_CHUNK,), jnp.int32),
            pltpu.VMEM((_CHUNK,), jnp.int32),
            pltpu.VMEM((_CHUNK,), jnp.int32),
            pltpu.VMEM((_CHUNK,), jnp.int32),
            pltpu.SemaphoreType.DMA,
            pltpu.SemaphoreType.DMA,
            pltpu.SemaphoreType.DMA,
            pltpu.SemaphoreType.DMA,
        ],
        compiler_params=pltpu.CompilerParams(needs_layout_passes=False),
    )(in32, tab32)
    out2d = (out32.reshape(_HIST // 8, _BATCH // 128, 8, 128)
             .transpose(1, 3, 0, 2).reshape(_BATCH, _HIST))
    outu = lax.bitcast_convert_type(out2d, jnp.uint32).astype(jnp.uint64)
    return lax.bitcast_convert_type(outu, jnp.int64)
